# Initial kernel scaffold; baseline (speedup 1.0000x reference)
import jax
import jax.numpy as jnp
from jax.experimental import pallas as pl


def kernel(pred_logits, pred_boxes, target_sizes):
    raise NotImplementedError("write your pallas kernel here")


# speedup vs baseline: 4.8348x; 4.8348x over previous
import functools

import jax
import jax.numpy as jnp
import numpy as np
from jax import lax
from jax.experimental import pallas as pl
from jax.experimental.pallas import tpu as pltpu
from jax.experimental.pallas import tpu_sc as plsc

NUM_SEL = 300
B = 16
N_BOX = 5000
N_CLS = 91
TOT = N_BOX * N_CLS
L = 16

WIN = 4096
NFULL = TOT // WIN
TAIL = TOT - NFULL * WIN
TAIL_P = 384
TAIL_V = TAIL_P // L
VPW = WIN // L

NBINS = 2048
CAP = 4096
OUTP = 384
BOXW = 20096

_NC = 2


def _sc_body(x_hbm, xtail_hbm, boxes_hbm, ts_hbm, scores_hbm, labels_hbm,
             boxesf_hbm,
             xwin, hist, cval, cidx, pk_a, pi_a, pk_b, pi_b,
             boxes_vm, sc_buf, lb_buf, bx_buf, ts_vm, sem_box):
    c = lax.axis_index("c")
    s = lax.axis_index("s")
    wid = s * _NC + c

    @pl.when(wid < B)
    def _():
        img = wid
        lanes = lax.iota(jnp.int32, L)
        ones_i = jnp.full((L,), 1, dtype=jnp.int32)
        sign_mask = jnp.int32(0x7FFFFFFF)

        pltpu.sync_copy(ts_hbm, ts_vm.at[pl.ds(0, 128)])
        box_dma = pltpu.async_copy(boxes_hbm.at[img], boxes_vm, sem_box)

        def f2key(v):
            bits = lax.bitcast_convert_type(v, jnp.int32)
            m = lax.shift_right_arithmetic(bits, 31)
            return lax.bitwise_xor(bits, lax.bitwise_and(m, sign_mask))

        def zero_body(k, _):
            hist[pl.ds(k * L, L)] = jnp.zeros((L,), jnp.int32)
            return 0
        lax.fori_loop(0, NBINS, zero_body, 0)

        def hist_vreg(j, _):
            v = xwin[pl.ds(j * L, L)]
            key = f2key(v)
            binv = lax.shift_right_arithmetic(key, 21) + 1024
            hidx = lax.shift_left(binv, 4) + lanes
            plsc.addupdate_scatter(hist, [hidx], ones_i)
            return 0

        def hist_window(w, _):
            pltpu.sync_copy(x_hbm.at[img, pl.ds(w * WIN, WIN)], xwin)
            lax.fori_loop(0, VPW, hist_vreg, 0)
            return 0
        lax.fori_loop(0, NFULL, hist_window, 0)

        pltpu.sync_copy(xtail_hbm.at[img], xwin.at[pl.ds(0, TAIL_P)])
        lax.fori_loop(0, TAIL_V, hist_vreg, 0)

        ngroups = NBINS // 16

        def grp_body(g, carry):
            cum, gstar, cum_above = carry
            g_rev = (ngroups - 1) - g
            acc = jnp.zeros((L,), jnp.int32)
            for b in range(16):
                acc = acc + hist[pl.ds(((g_rev * 16 + b) * L), L)]
            t = jnp.sum(acc)
            newcum = cum + t
            hit = jnp.logical_and(gstar < 0, newcum >= NUM_SEL)
            gstar = jnp.where(hit, g_rev, gstar)
            cum_above = jnp.where(hit, cum, cum_above)
            return newcum, gstar, cum_above

        _, gstar, cum_above_g = lax.fori_loop(
            0, ngroups, grp_body, (jnp.int32(0), jnp.int32(-1), jnp.int32(0)))

        def bin_body(k, carry):
            cum, bstar, cum_above = carry
            b_rev = 15 - k
            t = jnp.sum(hist[pl.ds((gstar * 16 + b_rev) * L, L)])
            newcum = cum + t
            hit = jnp.logical_and(bstar < 0, cum_above_g + newcum >= NUM_SEL)
            bstar = jnp.where(hit, b_rev, bstar)
            cum_above = jnp.where(hit, cum_above_g + cum, cum_above)
            return newcum, bstar, cum_above

        _, bstar, _ = lax.fori_loop(
            0, 16, bin_body, (jnp.int32(0), jnp.int32(-1), jnp.int32(0)))

        bin_g = gstar * 16 + bstar
        edge = lax.shift_left(bin_g - 1024, 21)
        int_min = jnp.int32(-2147483648)
        margin = jnp.int32(1 << 16)
        thr = jnp.where(edge <= int_min + margin, int_min, edge - margin)

        def gat_vreg_factory(base):
            def gat_vreg(j, cnt):
                v = xwin[pl.ds(j * L, L)]
                key = f2key(v)
                msk = key >= thr
                iv = lanes + (base + j * L)
                cc = jnp.minimum(cnt, CAP)
                plsc.store_compressed(cval.at[pl.ds(cc, L)], v, mask=msk)
                plsc.store_compressed(cidx.at[pl.ds(cc, L)], iv, mask=msk)
                return cnt + jnp.sum(msk.astype(jnp.int32))
            return gat_vreg

        def gat_window(w, cnt):
            pltpu.sync_copy(x_hbm.at[img, pl.ds(w * WIN, WIN)], xwin)

            def gat_vreg(j, cnt):
                v = xwin[pl.ds(j * L, L)]
                key = f2key(v)
                msk = key >= thr
                iv = lanes + (w * WIN + j * L)
                cc = jnp.minimum(cnt, CAP)
                plsc.store_compressed(cval.at[pl.ds(cc, L)], v, mask=msk)
                plsc.store_compressed(cidx.at[pl.ds(cc, L)], iv, mask=msk)
                return cnt + jnp.sum(msk.astype(jnp.int32))

            return lax.fori_loop(0, VPW, gat_vreg, cnt)

        cnt = lax.fori_loop(0, NFULL, gat_window, jnp.int32(0))

        pltpu.sync_copy(xtail_hbm.at[img], xwin.at[pl.ds(0, TAIL_P)])
        cnt = lax.fori_loop(0, TAIL_V, gat_vreg_factory(NFULL * WIN), cnt)

        n = jnp.minimum(cnt, jnp.int32(CAP))
        nv = lax.shift_right_arithmetic(n + 15, 4)

        def prob_body(k, _):
            v = cval[pl.ds(k * L, L)]
            p = 1.0 / (1.0 + jnp.exp(-v))
            kkey = lax.bitcast_convert_type(p, jnp.int32)
            rem = n - k * L
            mk = lanes < rem
            pk_a[pl.ds(k * L, L)] = jnp.where(mk, kkey, 0)
            pi_a[pl.ds(k * L, L)] = jnp.where(mk, cidx[pl.ds(k * L, L)], 0)
            return 0
        lax.fori_loop(0, nv, prob_body, 0)

        for t in range(32):
            if t % 2 == 0:
                src_k, src_i, dst_k, dst_i = pk_a, pi_a, pk_b, pi_b
            else:
                src_k, src_i, dst_k, dst_i = pk_b, pi_b, pk_a, pi_a
            bitmask = jnp.int32(np.int32(np.uint32(1 << t)))

            def cnt_body(k, c1):
                kk = src_k[pl.ds(k * L, L)]
                mb = lax.bitwise_and(kk, bitmask) != 0
                return c1 + jnp.sum(mb.astype(jnp.int32))
            n_ones = lax.fori_loop(0, nv, cnt_body, jnp.int32(0))

            def part_body(k, carry):
                oc, zc = carry
                kk = src_k[pl.ds(k * L, L)]
                ii = src_i[pl.ds(k * L, L)]
                mb = lax.bitwise_and(kk, bitmask) != 0
                mz = jnp.logical_not(mb)
                plsc.store_compressed(dst_k.at[pl.ds(oc, L)], kk, mask=mb)
                plsc.store_compressed(dst_i.at[pl.ds(oc, L)], ii, mask=mb)
                plsc.store_compressed(dst_k.at[pl.ds(zc, L)], kk, mask=mz)
                plsc.store_compressed(dst_i.at[pl.ds(zc, L)], ii, mask=mz)
                no = jnp.sum(mb.astype(jnp.int32))
                return oc + no, zc + (L - no)
            lax.fori_loop(0, nv, part_body, (jnp.int32(0), n_ones))

        box_dma.wait()
        tsv = ts_vm[pl.ds(img * 2, L)]
        fh = tsv[0].astype(jnp.float32)
        fw = tsv[1].astype(jnp.float32)
        inv91 = jnp.float32(1.0 / 91.0)

        for j in range(OUTP // L):
            kk = pk_a[pl.ds(j * L, L)]
            sc_buf[pl.ds(j * L, L)] = lax.bitcast_convert_type(kk, jnp.float32)
            iv = pi_a[pl.ds(j * L, L)]
            fidx = iv.astype(jnp.float32)
            bidx = ((fidx + 0.5) * inv91).astype(jnp.int32)
            lbl = iv - bidx * 91
            bidx = jnp.clip(bidx, 0, N_BOX - 1)
            lb_buf[pl.ds(j * L, L)] = jnp.clip(lbl, 0, N_CLS - 1)
            b4 = lax.shift_left(bidx, 2)
            cx = plsc.load_gather(boxes_vm, [b4])
            cy = plsc.load_gather(boxes_vm, [b4 + 1])
            w_ = plsc.load_gather(boxes_vm, [b4 + 2])
            h_ = plsc.load_gather(boxes_vm, [b4 + 3])
            x0 = (cx - 0.5 * w_) * fw
            y0 = (cy - 0.5 * h_) * fh
            x1 = (cx + 0.5 * w_) * fw
            y1 = (cy + 0.5 * h_) * fh
            pos = lax.shift_left(lanes + j * L, 2)
            plsc.store_scatter(bx_buf, [pos], x0)
            plsc.store_scatter(bx_buf, [pos + 1], y0)
            plsc.store_scatter(bx_buf, [pos + 2], x1)
            plsc.store_scatter(bx_buf, [pos + 3], y1)

        pltpu.sync_copy(sc_buf, scores_hbm.at[img])
        pltpu.sync_copy(lb_buf, labels_hbm.at[img])
        pltpu.sync_copy(bx_buf, boxesf_hbm.at[img])


@jax.jit
def _postprocess_sc(x, xtail, boxes, ts):
    mesh = plsc.VectorSubcoreMesh(core_axis_name="c", subcore_axis_name="s")
    f = pl.kernel(
        _sc_body,
        out_type=(
            jax.ShapeDtypeStruct((B, OUTP), jnp.float32),
            jax.ShapeDtypeStruct((B, OUTP), jnp.int32),
            jax.ShapeDtypeStruct((B, OUTP * 4), jnp.float32),
        ),
        mesh=mesh,
        compiler_params=pltpu.CompilerParams(needs_layout_passes=False),
        scratch_types=[
            pltpu.VMEM((WIN,), jnp.float32),
            pltpu.VMEM((NBINS * L,), jnp.int32),
            pltpu.VMEM((CAP + L,), jnp.float32),
            pltpu.VMEM((CAP + L,), jnp.int32),
            pltpu.VMEM((CAP + L,), jnp.int32),
            pltpu.VMEM((CAP + L,), jnp.int32),
            pltpu.VMEM((CAP + L,), jnp.int32),
            pltpu.VMEM((CAP + L,), jnp.int32),
            pltpu.VMEM((BOXW,), jnp.float32),
            pltpu.VMEM((OUTP,), jnp.float32),
            pltpu.VMEM((OUTP,), jnp.int32),
            pltpu.VMEM((OUTP * 4,), jnp.float32),
            pltpu.VMEM((128 + L,), jnp.int32),
            pltpu.SemaphoreType.DMA,
        ],
    )
    return f(x, xtail, boxes, ts)


def kernel(pred_logits, pred_boxes, target_sizes):
    x = pred_logits.reshape(B, TOT)
    xtail = jnp.pad(x[:, NFULL * WIN:], ((0, 0), (0, TAIL_P - TAIL)),
                    constant_values=-jnp.inf)
    boxes_p = jnp.pad(pred_boxes.reshape(B, N_BOX * 4),
                      ((0, 0), (0, BOXW - N_BOX * 4)))
    ts = jnp.pad(target_sizes.astype(jnp.int32).reshape(2 * B), (0, 128 - 2 * B))
    scores_p, labels_p, boxes_f = _postprocess_sc(x, xtail, boxes_p, ts)
    return (scores_p[:, :NUM_SEL], labels_p[:, :NUM_SEL],
            boxes_f.reshape(B, OUTP, 4)[:, :NUM_SEL])


# speedup vs baseline: 7.6656x; 1.5855x over previous
import jax
import jax.numpy as jnp
import numpy as np
from jax import lax
from jax.experimental import pallas as pl
from jax.experimental.pallas import tpu as pltpu
from jax.experimental.pallas import tpu_sc as plsc

NUM_SEL = 300
B = 16
N_BOX = 5000
N_CLS = 91
TOT = N_BOX * N_CLS
L = 16

WIN = 4096
NFULL = TOT // WIN
TAIL = TOT - NFULL * WIN
TAIL_P = 384
VPW = WIN // L
U = 8

NBINS = 2048
CAP = 4096
OUTP = 384
BOXW = 20096

_NC = 2


def _sc_body(x_hbm, xtail_hbm, boxes_hbm, ts_hbm, scores_hbm, labels_hbm,
             boxesf_hbm,
             xwin, hist, cval, cidx, pk_a, pi_a, pk_b, pi_b,
             boxes_vm, sc_buf, lb_buf, bx_buf, ts_vm, sem_box, sem_win):
    c = lax.axis_index("c")
    s = lax.axis_index("s")
    wid = s * _NC + c

    @pl.when(wid < B)
    def _():
        img = wid
        lanes = lax.iota(jnp.int32, L)
        ones_i = jnp.full((L,), 1, dtype=jnp.int32)
        sign_mask = jnp.int32(0x7FFFFFFF)

        pltpu.sync_copy(ts_hbm, ts_vm.at[pl.ds(0, 128)])
        box_dma = pltpu.async_copy(boxes_hbm.at[img], boxes_vm, sem_box)

        def f2key(v):
            bits = lax.bitcast_convert_type(v, jnp.int32)
            m = lax.shift_right_arithmetic(bits, 31)
            return lax.bitwise_xor(bits, lax.bitwise_and(m, sign_mask))

        def start_win(w, slot):
            pltpu.async_copy(x_hbm.at[img, pl.ds(w * WIN, WIN)],
                             xwin.at[slot], sem_win.at[slot])

        def wait_win(slot):
            pltpu.make_async_copy(x_hbm.at[img, pl.ds(0, WIN)],
                                  xwin.at[slot], sem_win.at[slot]).wait()

        def stream_image(process8, init):
            start_win(0, 0)

            def win_body(w, carry):
                slot = lax.rem(w, 2)

                @pl.when(w + 1 < NFULL)
                def _():
                    start_win(w + 1, 1 - slot)

                wait_win(slot)
                base = w * WIN

                def blk(jj, c2):
                    return process8(slot, base, jj, c2)
                return lax.fori_loop(0, VPW // U, blk, carry)

            carry = lax.fori_loop(0, NFULL, win_body, init)

            pltpu.sync_copy(xtail_hbm.at[img], xwin.at[0, pl.ds(0, TAIL_P)])

            def blkt(jj, c2):
                return process8(0, NFULL * WIN, jj, c2)
            return lax.fori_loop(0, TAIL_P // (L * U), blkt, carry)

        def zero_body(k, _):
            for u in range(U):
                hist[pl.ds((k * U + u) * L, L)] = jnp.zeros((L,), jnp.int32)
            return 0
        lax.fori_loop(0, NBINS // U, zero_body, 0)

        def hist8(slot, base, jj, carry):
            for u in range(U):
                v = xwin[slot, pl.ds((jj * U + u) * L, L)]
                key = f2key(v)
                binv = lax.shift_right_arithmetic(key, 21) + 1024
                hidx = lax.shift_left(binv, 4) + lanes
                plsc.addupdate_scatter(hist, [hidx], ones_i)
            return carry

        stream_image(hist8, 0)

        ngroups = NBINS // 16

        def grp_body(g, carry):
            cum, gstar, cum_above = carry
            g_rev = (ngroups - 1) - g
            acc = jnp.zeros((L,), jnp.int32)
            for b in range(16):
                acc = acc + hist[pl.ds(((g_rev * 16 + b) * L), L)]
            t = jnp.sum(acc)
            newcum = cum + t
            hit = jnp.logical_and(gstar < 0, newcum >= NUM_SEL)
            gstar = jnp.where(hit, g_rev, gstar)
            cum_above = jnp.where(hit, cum, cum_above)
            return newcum, gstar, cum_above

        _, gstar, cum_above_g = lax.fori_loop(
            0, ngroups, grp_body, (jnp.int32(0), jnp.int32(-1), jnp.int32(0)))

        def bin_body(k, carry):
            cum, bstar, cum_above = carry
            b_rev = 15 - k
            t = jnp.sum(hist[pl.ds((gstar * 16 + b_rev) * L, L)])
            newcum = cum + t
            hit = jnp.logical_and(bstar < 0, cum_above_g + newcum >= NUM_SEL)
            bstar = jnp.where(hit, b_rev, bstar)
            cum_above = jnp.where(hit, cum_above_g + cum, cum_above)
            return newcum, bstar, cum_above

        _, bstar, _ = lax.fori_loop(
            0, 16, bin_body, (jnp.int32(0), jnp.int32(-1), jnp.int32(0)))

        bin_g = gstar * 16 + bstar
        edge = lax.shift_left(bin_g - 1024, 21)
        int_min = jnp.int32(-2147483648)
        margin = jnp.int32(1 << 16)
        thr = jnp.where(edge <= int_min + margin, int_min, edge - margin)

        def gat8(slot, base, jj, cnt):
            vs, ivs, ms, cs = [], [], [], []
            for u in range(U):
                j = jj * U + u
                v = xwin[slot, pl.ds(j * L, L)]
                key = f2key(v)
                m = key >= thr
                iv = lanes + (base + j * L)
                vs.append(v)
                ivs.append(iv)
                ms.append(m)
                cs.append(jnp.sum(m.astype(jnp.int32)))
            off = cnt
            for u in range(U):
                cc = jnp.minimum(off, CAP)
                plsc.store_compressed(cval.at[pl.ds(cc, L)], vs[u], mask=ms[u])
                plsc.store_compressed(cidx.at[pl.ds(cc, L)], ivs[u],
                                      mask=ms[u])
                off = off + cs[u]
            return off

        cnt = stream_image(gat8, jnp.int32(0))

        n = jnp.minimum(cnt, jnp.int32(CAP))
        nv = lax.shift_right_arithmetic(n + 15, 4)

        def prob_body(k, _):
            v = cval[pl.ds(k * L, L)]
            p = 1.0 / (1.0 + jnp.exp(-v))
            kkey = lax.bitcast_convert_type(p, jnp.int32)
            rem = n - k * L
            mk = lanes < rem
            pk_a[pl.ds(k * L, L)] = jnp.where(mk, kkey, 0)
            pi_a[pl.ds(k * L, L)] = jnp.where(mk, cidx[pl.ds(k * L, L)], 0)
            return 0
        lax.fori_loop(0, nv, prob_body, 0)

        for t in range(32):
            if t % 2 == 0:
                src_k, src_i, dst_k, dst_i = pk_a, pi_a, pk_b, pi_b
            else:
                src_k, src_i, dst_k, dst_i = pk_b, pi_b, pk_a, pi_a
            bitmask = jnp.int32(np.int32(np.uint32(1 << t)))

            def cnt_body(k, c1):
                kk = src_k[pl.ds(k * L, L)]
                mb = lax.bitwise_and(kk, bitmask) != 0
                return c1 + jnp.sum(mb.astype(jnp.int32))
            n_ones = lax.fori_loop(0, nv, cnt_body, jnp.int32(0))

            def part_body(k, carry):
                oc, zc = carry
                kk = src_k[pl.ds(k * L, L)]
                ii = src_i[pl.ds(k * L, L)]
                mb = lax.bitwise_and(kk, bitmask) != 0
                mz = jnp.logical_not(mb)
                plsc.store_compressed(dst_k.at[pl.ds(oc, L)], kk, mask=mb)
                plsc.store_compressed(dst_i.at[pl.ds(oc, L)], ii, mask=mb)
                plsc.store_compressed(dst_k.at[pl.ds(zc, L)], kk, mask=mz)
                plsc.store_compressed(dst_i.at[pl.ds(zc, L)], ii, mask=mz)
                no = jnp.sum(mb.astype(jnp.int32))
                return oc + no, zc + (L - no)
            lax.fori_loop(0, nv, part_body, (jnp.int32(0), n_ones))

        box_dma.wait()
        tsv = ts_vm[pl.ds(img * 2, L)]
        fh = tsv[0].astype(jnp.float32)
        fw = tsv[1].astype(jnp.float32)
        inv91 = jnp.float32(1.0 / 91.0)

        for j in range(OUTP // L):
            kk = pk_a[pl.ds(j * L, L)]
            sc_buf[pl.ds(j * L, L)] = lax.bitcast_convert_type(kk,
                                                               jnp.float32)
            iv = pi_a[pl.ds(j * L, L)]
            fidx = iv.astype(jnp.float32)
            bidx = ((fidx + 0.5) * inv91).astype(jnp.int32)
            lbl = iv - bidx * 91
            bidx = jnp.clip(bidx, 0, N_BOX - 1)
            lb_buf[pl.ds(j * L, L)] = jnp.clip(lbl, 0, N_CLS - 1)
            b4 = lax.shift_left(bidx, 2)
            cx = plsc.load_gather(boxes_vm, [b4])
            cy = plsc.load_gather(boxes_vm, [b4 + 1])
            w_ = plsc.load_gather(boxes_vm, [b4 + 2])
            h_ = plsc.load_gather(boxes_vm, [b4 + 3])
            x0 = (cx - 0.5 * w_) * fw
            y0 = (cy - 0.5 * h_) * fh
            x1 = (cx + 0.5 * w_) * fw
            y1 = (cy + 0.5 * h_) * fh
            pos = lax.shift_left(lanes + j * L, 2)
            plsc.store_scatter(bx_buf, [pos], x0)
            plsc.store_scatter(bx_buf, [pos + 1], y0)
            plsc.store_scatter(bx_buf, [pos + 2], x1)
            plsc.store_scatter(bx_buf, [pos + 3], y1)

        pltpu.sync_copy(sc_buf, scores_hbm.at[img])
        pltpu.sync_copy(lb_buf, labels_hbm.at[img])
        pltpu.sync_copy(bx_buf, boxesf_hbm.at[img])


@jax.jit
def _postprocess_sc(x, xtail, boxes, ts):
    mesh = plsc.VectorSubcoreMesh(core_axis_name="c", subcore_axis_name="s")
    f = pl.kernel(
        _sc_body,
        out_type=(
            jax.ShapeDtypeStruct((B, OUTP), jnp.float32),
            jax.ShapeDtypeStruct((B, OUTP), jnp.int32),
            jax.ShapeDtypeStruct((B, OUTP * 4), jnp.float32),
        ),
        mesh=mesh,
        compiler_params=pltpu.CompilerParams(needs_layout_passes=False),
        scratch_types=[
            pltpu.VMEM((2, WIN), jnp.float32),
            pltpu.VMEM((NBINS * L,), jnp.int32),
            pltpu.VMEM((CAP + L,), jnp.float32),
            pltpu.VMEM((CAP + L,), jnp.int32),
            pltpu.VMEM((CAP + L,), jnp.int32),
            pltpu.VMEM((CAP + L,), jnp.int32),
            pltpu.VMEM((CAP + L,), jnp.int32),
            pltpu.VMEM((CAP + L,), jnp.int32),
            pltpu.VMEM((BOXW,), jnp.float32),
            pltpu.VMEM((OUTP,), jnp.float32),
            pltpu.VMEM((OUTP,), jnp.int32),
            pltpu.VMEM((OUTP * 4,), jnp.float32),
            pltpu.VMEM((128 + L,), jnp.int32),
            pltpu.SemaphoreType.DMA,
            pltpu.SemaphoreType.DMA((2,)),
        ],
    )
    return f(x, xtail, boxes, ts)


def kernel(pred_logits, pred_boxes, target_sizes):
    x = pred_logits.reshape(B, TOT)
    xtail = jnp.pad(x[:, NFULL * WIN:], ((0, 0), (0, TAIL_P - TAIL)),
                    constant_values=-jnp.inf)
    boxes_p = jnp.pad(pred_boxes.reshape(B, N_BOX * 4),
                      ((0, 0), (0, BOXW - N_BOX * 4)))
    ts = jnp.pad(target_sizes.astype(jnp.int32).reshape(2 * B),
                 (0, 128 - 2 * B))
    scores_p, labels_p, boxes_f = _postprocess_sc(x, xtail, boxes_p, ts)
    return (scores_p[:, :NUM_SEL], labels_p[:, :NUM_SEL],
            boxes_f.reshape(B, OUTP, 4)[:, :NUM_SEL])


# speedup vs baseline: 10.8640x; 1.4172x over previous
import jax
import jax.numpy as jnp
import numpy as np
from jax import lax
from jax.experimental import pallas as pl
from jax.experimental.pallas import tpu as pltpu
from jax.experimental.pallas import tpu_sc as plsc

NUM_SEL = 300
B = 16
N_BOX = 5000
N_CLS = 91
TOT = N_BOX * N_CLS
L = 16

WIN = 4096
NFULL = TOT // WIN
TAIL = TOT - NFULL * WIN
TAIL_P = 384
VPW = WIN // L
U = 8
W_H0 = 56

NBINS = 2048
CAP = 4096
CAPB = 4224
OUTP = 384
BOXW = 20096

_NC = 2
IPC = 8


def _sc_body(x_hbm, xtail_hbm, boxes_hbm, ts_hbm, scores_hbm, labels_hbm,
             boxesf_hbm,
             xwin, hist, hchunk, cval, cidx, pk_a, pi_a, pk_b, pi_b,
             boxes_vm, sc_buf, lb_buf, bx_buf, ts_vm, thrbuf,
             sh_hist, sh_cv, sh_ci, sh_thr, sh_cnt,
             sem_box, sem_win):
    c = lax.axis_index("c")
    s = lax.axis_index("s")
    half = lax.div(s, IPC)
    img_local = lax.rem(s, IPC)
    img = c * IPC + img_local

    lanes = lax.iota(jnp.int32, L)
    ones_i = jnp.full((L,), 1, dtype=jnp.int32)
    sign_mask = jnp.int32(0x7FFFFFFF)

    w_lo = half * W_H0
    w_hi = W_H0 + half * (NFULL - W_H0)

    @pl.when(half == 0)
    def _():
        pltpu.sync_copy(ts_hbm, ts_vm.at[pl.ds(0, 128)])
        pltpu.async_copy(boxes_hbm.at[img], boxes_vm, sem_box)

    def f2key(v):
        bits = lax.bitcast_convert_type(v, jnp.int32)
        m = lax.shift_right_arithmetic(bits, 31)
        return lax.bitwise_xor(bits, lax.bitwise_and(m, sign_mask))

    def start_win(w, slot):
        pltpu.async_copy(x_hbm.at[img, pl.ds(w * WIN, WIN)],
                         xwin.at[slot], sem_win.at[slot])

    def wait_win(slot):
        pltpu.make_async_copy(x_hbm.at[img, pl.ds(0, WIN)],
                              xwin.at[slot], sem_win.at[slot]).wait()

    def stream_half(process8, init):
        start_win(w_lo, 0)

        def win_body(w, carry):
            slot = lax.rem(w - w_lo, 2)

            @pl.when(w + 1 < w_hi)
            def _():
                start_win(w + 1, 1 - slot)

            wait_win(slot)
            base = w * WIN

            def blk(jj, c2):
                return process8(slot, base, jj, c2)
            return lax.fori_loop(0, VPW // U, blk, carry)

        carry = lax.fori_loop(w_lo, w_hi, win_body, init)

        def tail_fn(carry):
            pltpu.sync_copy(xtail_hbm.at[img], xwin.at[0, pl.ds(0, TAIL_P)])

            def blkt(jj, c2):
                return process8(0, NFULL * WIN, jj, c2)
            return lax.fori_loop(0, TAIL_P // (L * U), blkt, carry)

        return lax.cond(half == 1, tail_fn, lambda cc: cc, carry)

    def zero_body(k, _):
        for u in range(U):
            hist[pl.ds((k * U + u) * L, L)] = jnp.zeros((L,), jnp.int32)
        return 0
    lax.fori_loop(0, NBINS // U, zero_body, 0)

    def hist8(slot, base, jj, carry):
        for u in range(U):
            v = xwin[slot, pl.ds((jj * U + u) * L, L)]
            key = f2key(v)
            binv = lax.shift_right_arithmetic(key, 21) + 1024
            hidx = lax.shift_left(binv, 4) + lanes
            plsc.addupdate_scatter(hist, [hidx], ones_i)
        return carry

    stream_half(hist8, 0)

    @pl.when(half == 1)
    def _():
        pltpu.sync_copy(hist, sh_hist.at[img_local])
    plsc.subcore_barrier()

    @pl.when(half == 0)
    def _():
        ngroups = NBINS // 16
        gr_per_ch = 8

        def chunk_body(chk, carry):
            ch_rev = 15 - chk
            pltpu.sync_copy(sh_hist.at[img_local,
                                       pl.ds(ch_rev * 2048, 2048)], hchunk)

            def grp_body(g, carry2):
                cum, gstar, cum_above = carry2
                g_rev = ch_rev * gr_per_ch + (gr_per_ch - 1) - g
                acc = jnp.zeros((L,), jnp.int32)
                for b in range(16):
                    o = (g_rev * 16 + b) * L
                    oc = o - ch_rev * 2048
                    acc = acc + hist[pl.ds(o, L)] + hchunk[pl.ds(oc, L)]
                t = jnp.sum(acc)
                newcum = cum + t
                hit = jnp.logical_and(gstar < 0, newcum >= NUM_SEL)
                gstar = jnp.where(hit, g_rev, gstar)
                cum_above = jnp.where(hit, cum, cum_above)
                return newcum, gstar, cum_above

            return lax.fori_loop(0, gr_per_ch, grp_body, carry)

        _, gstar, cum_above_g = lax.fori_loop(
            0, 16, chunk_body,
            (jnp.int32(0), jnp.int32(-1), jnp.int32(0)))

        gchunk = lax.shift_right_arithmetic(gstar, 3)
        pltpu.sync_copy(sh_hist.at[img_local, pl.ds(gchunk * 2048, 2048)],
                        hchunk)

        def bin_body(k, carry):
            cum, bstar, cum_above = carry
            b_rev = 15 - k
            o = (gstar * 16 + b_rev) * L
            oc = o - gchunk * 2048
            t = jnp.sum(hist[pl.ds(o, L)] + hchunk[pl.ds(oc, L)])
            newcum = cum + t
            hit = jnp.logical_and(bstar < 0, cum_above_g + newcum >= NUM_SEL)
            bstar = jnp.where(hit, b_rev, bstar)
            cum_above = jnp.where(hit, cum_above_g + cum, cum_above)
            return newcum, bstar, cum_above

        _, bstar, _ = lax.fori_loop(
            0, 16, bin_body, (jnp.int32(0), jnp.int32(-1), jnp.int32(0)))

        bin_g = gstar * 16 + bstar
        edge = lax.shift_left(bin_g - 1024, 21)
        int_min = jnp.int32(-2147483648)
        margin = jnp.int32(1 << 16)
        thr0 = jnp.where(edge <= int_min + margin, int_min, edge - margin)
        thrbuf[pl.ds(0, L)] = jnp.full((L,), 0, jnp.int32) + thr0
        pltpu.sync_copy(thrbuf, sh_thr.at[img_local])

    plsc.subcore_barrier()
    pltpu.sync_copy(sh_thr.at[img_local], thrbuf)
    thr = thrbuf[pl.ds(0, L)][0]

    def gat8(slot, base, jj, cnt):
        vs, ivs, ms = [], [], []
        for u in range(U):
            j = jj * U + u
            v = xwin[slot, pl.ds(j * L, L)]
            key = f2key(v)
            m = key >= thr
            iv = lanes + (base + j * L)
            vs.append(v)
            ivs.append(iv)
            ms.append(m)
        anym = ms[0]
        for u in range(1, U):
            anym = jnp.logical_or(anym, ms[u])
        nblk = jnp.sum(anym.astype(jnp.int32))

        def store_path(cnt_in):
            off = cnt_in
            for u in range(U):
                cc = jnp.minimum(off, CAP)
                plsc.store_compressed(cval.at[pl.ds(cc, L)], vs[u],
                                      mask=ms[u])
                plsc.store_compressed(cidx.at[pl.ds(cc, L)], ivs[u],
                                      mask=ms[u])
                off = off + jnp.sum(ms[u].astype(jnp.int32))
            return off

        return lax.cond(nblk > 0, store_path, lambda cc: cc, cnt)

    cnt = stream_half(gat8, jnp.int32(0))
    cnt = jnp.minimum(cnt, jnp.int32(CAP))

    @pl.when(half == 1)
    def _():
        pltpu.sync_copy(cval, sh_cv.at[img_local])
        pltpu.sync_copy(cidx, sh_ci.at[img_local])
        thrbuf[pl.ds(0, L)] = jnp.full((L,), 0, jnp.int32) + cnt
        pltpu.sync_copy(thrbuf, sh_cnt.at[img_local])
    plsc.subcore_barrier()

    @pl.when(half == 0)
    def _():
        pltpu.sync_copy(sh_cnt.at[img_local], thrbuf)
        n1 = thrbuf[pl.ds(0, L)][0]

        n0 = cnt
        n = jnp.minimum(n0 + n1, jnp.int32(CAP))
        m1 = n - n0
        nv0 = lax.shift_right_arithmetic(n0 + 15, 4)
        nv1 = lax.shift_right_arithmetic(m1 + 15, 4)
        nv = lax.shift_right_arithmetic(n + 15, 4)

        def sigkey(v):
            p = 1.0 / (1.0 + jnp.exp(-v))
            return lax.bitcast_convert_type(p, jnp.int32)

        def prob_body(k, _):
            v = cval[pl.ds(k * L, L)]
            kkey = sigkey(v)
            rem = n0 - k * L
            mk = lanes < rem
            pk_a[pl.ds(k * L, L)] = jnp.where(mk, kkey, 0)
            pi_a[pl.ds(k * L, L)] = jnp.where(mk, cidx[pl.ds(k * L, L)], 0)
            return 0
        lax.fori_loop(0, nv0, prob_body, 0)

        pltpu.sync_copy(sh_cv.at[img_local], cval)
        pltpu.sync_copy(sh_ci.at[img_local], cidx)

        def prob2_body(k, _):
            v = cval[pl.ds(k * L, L)]
            kkey = sigkey(v)
            rem = m1 - k * L
            mk = lanes < rem
            cc = n0 + k * L
            plsc.store_compressed(pk_a.at[pl.ds(cc, L)],
                                  jnp.where(mk, kkey, 0), mask=mk)
            plsc.store_compressed(pi_a.at[pl.ds(cc, L)],
                                  jnp.where(mk, cidx[pl.ds(k * L, L)], 0),
                                  mask=mk)
            return 0
        lax.fori_loop(0, nv1, prob2_body, 0)

        pk_a[pl.ds(n, L)] = jnp.zeros((L,), jnp.int32)
        pi_a[pl.ds(n, L)] = jnp.zeros((L,), jnp.int32)

        for t in range(32):
            if t % 2 == 0:
                src_k, src_i, dst_k, dst_i = pk_a, pi_a, pk_b, pi_b
            else:
                src_k, src_i, dst_k, dst_i = pk_b, pi_b, pk_a, pi_a
            bitmask = jnp.int32(np.int32(np.uint32(1 << t)))

            def cnt_body(k, c1):
                kk = src_k[pl.ds(k * L, L)]
                mb = lax.bitwise_and(kk, bitmask) != 0
                return c1 + mb.astype(jnp.int32)
            ones_vec = lax.fori_loop(0, nv, cnt_body,
                                     jnp.zeros((L,), jnp.int32))
            n_ones = jnp.sum(ones_vec)

            def part_body(k, carry):
                oc, zc = carry
                kk = src_k[pl.ds(k * L, L)]
                ii = src_i[pl.ds(k * L, L)]
                mb = lax.bitwise_and(kk, bitmask) != 0
                mz = jnp.logical_not(mb)
                plsc.store_compressed(dst_k.at[pl.ds(oc, L)], kk, mask=mb)
                plsc.store_compressed(dst_i.at[pl.ds(oc, L)], ii, mask=mb)
                plsc.store_compressed(dst_k.at[pl.ds(zc, L)], kk, mask=mz)
                plsc.store_compressed(dst_i.at[pl.ds(zc, L)], ii, mask=mz)
                no = jnp.sum(mb.astype(jnp.int32))
                return oc + no, zc + (L - no)
            lax.fori_loop(0, nv, part_body, (jnp.int32(0), n_ones))

        pltpu.make_async_copy(boxes_hbm.at[img], boxes_vm, sem_box).wait()
        tsv = ts_vm[pl.ds(img * 2, L)]
        fh = tsv[0].astype(jnp.float32)
        fw = tsv[1].astype(jnp.float32)
        inv91 = jnp.float32(1.0 / 91.0)

        for j in range(OUTP // L):
            kk = pk_a[pl.ds(j * L, L)]
            sc_buf[pl.ds(j * L, L)] = lax.bitcast_convert_type(kk,
                                                               jnp.float32)
            iv = pi_a[pl.ds(j * L, L)]
            fidx = iv.astype(jnp.float32)
            bidx = ((fidx + 0.5) * inv91).astype(jnp.int32)
            lbl = iv - bidx * 91
            bidx = jnp.clip(bidx, 0, N_BOX - 1)
            lb_buf[pl.ds(j * L, L)] = jnp.clip(lbl, 0, N_CLS - 1)
            b4 = lax.shift_left(bidx, 2)
            cx = plsc.load_gather(boxes_vm, [b4])
            cy = plsc.load_gather(boxes_vm, [b4 + 1])
            w_ = plsc.load_gather(boxes_vm, [b4 + 2])
            h_ = plsc.load_gather(boxes_vm, [b4 + 3])
            x0 = (cx - 0.5 * w_) * fw
            y0 = (cy - 0.5 * h_) * fh
            x1 = (cx + 0.5 * w_) * fw
            y1 = (cy + 0.5 * h_) * fh
            pos = lax.shift_left(lanes + j * L, 2)
            plsc.store_scatter(bx_buf, [pos], x0)
            plsc.store_scatter(bx_buf, [pos + 1], y0)
            plsc.store_scatter(bx_buf, [pos + 2], x1)
            plsc.store_scatter(bx_buf, [pos + 3], y1)

        pltpu.sync_copy(sc_buf, scores_hbm.at[img])
        pltpu.sync_copy(lb_buf, labels_hbm.at[img])
        pltpu.sync_copy(bx_buf, boxesf_hbm.at[img])


@jax.jit
def _postprocess_sc(x, xtail, boxes, ts):
    mesh = plsc.VectorSubcoreMesh(core_axis_name="c", subcore_axis_name="s")
    f = pl.kernel(
        _sc_body,
        out_type=(
            jax.ShapeDtypeStruct((B, OUTP), jnp.float32),
            jax.ShapeDtypeStruct((B, OUTP), jnp.int32),
            jax.ShapeDtypeStruct((B, OUTP * 4), jnp.float32),
        ),
        mesh=mesh,
        compiler_params=pltpu.CompilerParams(needs_layout_passes=False),
        scratch_types=[
            pltpu.VMEM((2, WIN), jnp.float32),
            pltpu.VMEM((NBINS * L,), jnp.int32),
            pltpu.VMEM((2048,), jnp.int32),
            pltpu.VMEM((CAPB,), jnp.float32),
            pltpu.VMEM((CAPB,), jnp.int32),
            pltpu.VMEM((CAP + L,), jnp.int32),
            pltpu.VMEM((CAP + L,), jnp.int32),
            pltpu.VMEM((CAP + L,), jnp.int32),
            pltpu.VMEM((CAP + L,), jnp.int32),
            pltpu.VMEM((BOXW,), jnp.float32),
            pltpu.VMEM((OUTP,), jnp.float32),
            pltpu.VMEM((OUTP,), jnp.int32),
            pltpu.VMEM((OUTP * 4,), jnp.float32),
            pltpu.VMEM((128 + L,), jnp.int32),
            pltpu.VMEM((128,), jnp.int32),
            pltpu.VMEM_SHARED((IPC, NBINS * L), jnp.int32),
            pltpu.VMEM_SHARED((IPC, CAPB), jnp.float32),
            pltpu.VMEM_SHARED((IPC, CAPB), jnp.int32),
            pltpu.VMEM_SHARED((IPC, 128), jnp.int32),
            pltpu.VMEM_SHARED((IPC, 128), jnp.int32),
            pltpu.SemaphoreType.DMA,
            pltpu.SemaphoreType.DMA((2,)),
        ],
    )
    return f(x, xtail, boxes, ts)


def kernel(pred_logits, pred_boxes, target_sizes):
    x = pred_logits.reshape(B, TOT)
    xtail = jnp.pad(x[:, NFULL * WIN:], ((0, 0), (0, TAIL_P - TAIL)),
                    constant_values=-jnp.inf)
    boxes_p = jnp.pad(pred_boxes.reshape(B, N_BOX * 4),
                      ((0, 0), (0, BOXW - N_BOX * 4)))
    ts = jnp.pad(target_sizes.astype(jnp.int32).reshape(2 * B),
                 (0, 128 - 2 * B))
    scores_p, labels_p, boxes_f = _postprocess_sc(x, xtail, boxes_p, ts)
    return (scores_p[:, :NUM_SEL], labels_p[:, :NUM_SEL],
            boxes_f.reshape(B, OUTP, 4)[:, :NUM_SEL])


# speedup vs baseline: 11.1926x; 1.0302x over previous
import jax
import jax.numpy as jnp
import numpy as np
from jax import lax
from jax.experimental import pallas as pl
from jax.experimental.pallas import tpu as pltpu
from jax.experimental.pallas import tpu_sc as plsc

NUM_SEL = 300
B = 16
N_BOX = 5000
N_CLS = 91
TOT = N_BOX * N_CLS
L = 16

WIN = 4096
NFULL = TOT // WIN
TAIL = TOT - NFULL * WIN
TAIL_P = 384
VPW = WIN // L
U = 8
W_H0 = 56

NBINS = 2048
CAP = 4096
CAPB = 4224
OUTP = 384
BOXW = 20096

THR_SPEC = 0x40333333

_NC = 2
IPC = 8


def _sc_body(x_hbm, xtail_hbm, boxes_hbm, ts_hbm, scores_hbm, labels_hbm,
             boxesf_hbm,
             xwin, hist, hchunk, cval, cidx, pk_a, pi_a, pk_b, pi_b,
             boxes_vm, sc_buf, lb_buf, bx_buf, ts_vm, thrbuf,
             sh_hist, sh_cv, sh_ci, sh_thr, sh_cnt,
             sem_box, sem_win):
    c = lax.axis_index("c")
    s = lax.axis_index("s")
    half = lax.div(s, IPC)
    img_local = lax.rem(s, IPC)
    img = c * IPC + img_local

    lanes = lax.iota(jnp.int32, L)
    ones_i = jnp.full((L,), 1, dtype=jnp.int32)
    sign_mask = jnp.int32(0x7FFFFFFF)

    w_lo = half * W_H0
    w_hi = W_H0 + half * (NFULL - W_H0)

    @pl.when(half == 0)
    def _():
        pltpu.sync_copy(ts_hbm, ts_vm.at[pl.ds(0, 128)])
        pltpu.async_copy(boxes_hbm.at[img], boxes_vm, sem_box)

    def f2key(v):
        bits = lax.bitcast_convert_type(v, jnp.int32)
        m = lax.shift_right_arithmetic(bits, 31)
        return lax.bitwise_xor(bits, lax.bitwise_and(m, sign_mask))

    def start_win(w, slot):
        pltpu.async_copy(x_hbm.at[img, pl.ds(w * WIN, WIN)],
                         xwin.at[slot], sem_win.at[slot])

    def wait_win(slot):
        pltpu.make_async_copy(x_hbm.at[img, pl.ds(0, WIN)],
                              xwin.at[slot], sem_win.at[slot]).wait()

    def stream_half(process8, init):
        start_win(w_lo, 0)

        def win_body(w, carry):
            slot = lax.rem(w - w_lo, 2)

            @pl.when(w + 1 < w_hi)
            def _():
                start_win(w + 1, 1 - slot)

            wait_win(slot)
            base = w * WIN

            def blk(jj, c2):
                return process8(slot, base, jj, c2)
            return lax.fori_loop(0, VPW // U, blk, carry)

        carry = lax.fori_loop(w_lo, w_hi, win_body, init)

        def tail_fn(carry):
            pltpu.sync_copy(xtail_hbm.at[img], xwin.at[0, pl.ds(0, TAIL_P)])

            def blkt(jj, c2):
                return process8(0, NFULL * WIN, jj, c2)
            return lax.fori_loop(0, TAIL_P // (L * U), blkt, carry)

        return lax.cond(half == 1, tail_fn, lambda cc: cc, carry)

    def zero_body(k, _):
        for u in range(U):
            hist[pl.ds((k * U + u) * L, L)] = jnp.zeros((L,), jnp.int32)
        return 0
    lax.fori_loop(0, NBINS // U, zero_body, 0)

    def fuse8(slot, base, jj, cnt):
        vs, ivs, ms = [], [], []
        for u in range(U):
            j = jj * U + u
            v = xwin[slot, pl.ds(j * L, L)]
            key = f2key(v)
            binv = lax.shift_right_arithmetic(key, 21) + 1024
            hidx = lax.shift_left(binv, 4) + lanes
            plsc.addupdate_scatter(hist, [hidx], ones_i)
            m = key >= jnp.int32(THR_SPEC)
            vs.append(v)
            ivs.append(lanes + (base + j * L))
            ms.append(m)
        anym = ms[0]
        for u in range(1, U):
            anym = jnp.logical_or(anym, ms[u])
        nblk = jnp.sum(anym.astype(jnp.int32))

        def store_path(cnt_in):
            off = cnt_in
            for u in range(U):
                cc = jnp.minimum(off, CAP)
                plsc.store_compressed(cval.at[pl.ds(cc, L)], vs[u],
                                      mask=ms[u])
                plsc.store_compressed(cidx.at[pl.ds(cc, L)], ivs[u],
                                      mask=ms[u])
                off = off + jnp.sum(ms[u].astype(jnp.int32))
            return off

        return lax.cond(nblk > 0, store_path, lambda cc: cc, cnt)

    cnt_spec = stream_half(fuse8, jnp.int32(0))

    @pl.when(half == 1)
    def _():
        pltpu.sync_copy(hist, sh_hist.at[img_local])
        pltpu.sync_copy(cval, sh_cv.at[img_local])
        pltpu.sync_copy(cidx, sh_ci.at[img_local])
        thrbuf[pl.ds(0, L)] = jnp.full((L,), 0, jnp.int32) + cnt_spec
        pltpu.sync_copy(thrbuf, sh_cnt.at[img_local])
    plsc.subcore_barrier()

    @pl.when(half == 0)
    def _():
        ngroups = NBINS // 16
        gr_per_ch = 8

        def chunk_body(chk, carry):
            ch_rev = 15 - chk
            pltpu.sync_copy(sh_hist.at[img_local,
                                       pl.ds(ch_rev * 2048, 2048)], hchunk)

            def grp_body(g, carry2):
                cum, gstar, cum_above = carry2
                g_rev = ch_rev * gr_per_ch + (gr_per_ch - 1) - g
                acc = jnp.zeros((L,), jnp.int32)
                for b in range(16):
                    o = (g_rev * 16 + b) * L
                    oc = o - ch_rev * 2048
                    acc = acc + hist[pl.ds(o, L)] + hchunk[pl.ds(oc, L)]
                t = jnp.sum(acc)
                newcum = cum + t
                hit = jnp.logical_and(gstar < 0, newcum >= NUM_SEL)
                gstar = jnp.where(hit, g_rev, gstar)
                cum_above = jnp.where(hit, cum, cum_above)
                return newcum, gstar, cum_above

            return lax.fori_loop(0, gr_per_ch, grp_body, carry)

        _, gstar, cum_above_g = lax.fori_loop(
            0, 16, chunk_body,
            (jnp.int32(0), jnp.int32(-1), jnp.int32(0)))

        gchunk = lax.shift_right_arithmetic(gstar, 3)
        pltpu.sync_copy(sh_hist.at[img_local, pl.ds(gchunk * 2048, 2048)],
                        hchunk)

        def bin_body(k, carry):
            cum, bstar, cum_above = carry
            b_rev = 15 - k
            o = (gstar * 16 + b_rev) * L
            oc = o - gchunk * 2048
            t = jnp.sum(hist[pl.ds(o, L)] + hchunk[pl.ds(oc, L)])
            newcum = cum + t
            hit = jnp.logical_and(bstar < 0, cum_above_g + newcum >= NUM_SEL)
            bstar = jnp.where(hit, b_rev, bstar)
            cum_above = jnp.where(hit, cum_above_g + cum, cum_above)
            return newcum, bstar, cum_above

        _, bstar, _ = lax.fori_loop(
            0, 16, bin_body, (jnp.int32(0), jnp.int32(-1), jnp.int32(0)))

        bin_g = gstar * 16 + bstar
        edge = lax.shift_left(bin_g - 1024, 21)
        int_min = jnp.int32(-2147483648)
        margin = jnp.int32(1 << 16)
        thr0 = jnp.where(edge <= int_min + margin, int_min, edge - margin)
        pltpu.sync_copy(sh_cnt.at[img_local], thrbuf)
        pcnt = thrbuf[pl.ds(0, L)][0]
        bad = jnp.logical_or(thr0 < jnp.int32(THR_SPEC),
                             jnp.logical_or(cnt_spec > CAP, pcnt > CAP))
        fb0 = bad.astype(jnp.int32)
        tv = jnp.full((L,), 0, jnp.int32) + thr0
        thrbuf[pl.ds(0, L)] = jnp.where(lanes == 1, fb0, tv)
        pltpu.sync_copy(thrbuf, sh_thr.at[img_local])

    plsc.subcore_barrier()
    pltpu.sync_copy(sh_thr.at[img_local], thrbuf)
    tfv = thrbuf[pl.ds(0, L)]
    thr = tfv[0]
    fb = tfv[1]

    def gat8(slot, base, jj, cnt):
        vs, ivs, ms = [], [], []
        for u in range(U):
            j = jj * U + u
            v = xwin[slot, pl.ds(j * L, L)]
            key = f2key(v)
            m = key >= thr
            iv = lanes + (base + j * L)
            vs.append(v)
            ivs.append(iv)
            ms.append(m)
        anym = ms[0]
        for u in range(1, U):
            anym = jnp.logical_or(anym, ms[u])
        nblk = jnp.sum(anym.astype(jnp.int32))

        def store_path(cnt_in):
            off = cnt_in
            for u in range(U):
                cc = jnp.minimum(off, CAP)
                plsc.store_compressed(cval.at[pl.ds(cc, L)], vs[u],
                                      mask=ms[u])
                plsc.store_compressed(cidx.at[pl.ds(cc, L)], ivs[u],
                                      mask=ms[u])
                off = off + jnp.sum(ms[u].astype(jnp.int32))
            return off

        return lax.cond(nblk > 0, store_path, lambda cc: cc, cnt)

    def regather(_):
        return stream_half(gat8, jnp.int32(0))

    cnt = lax.cond(fb > 0, regather, lambda cc: cc, cnt_spec)
    cnt = jnp.minimum(cnt, jnp.int32(CAP))

    @pl.when(jnp.logical_and(half == 1, fb > 0))
    def _():
        pltpu.sync_copy(cval, sh_cv.at[img_local])
        pltpu.sync_copy(cidx, sh_ci.at[img_local])
        thrbuf[pl.ds(0, L)] = jnp.full((L,), 0, jnp.int32) + cnt
        pltpu.sync_copy(thrbuf, sh_cnt.at[img_local])
    plsc.subcore_barrier()

    @pl.when(half == 0)
    def _():
        pltpu.sync_copy(sh_cnt.at[img_local], thrbuf)
        n1 = jnp.minimum(thrbuf[pl.ds(0, L)][0], jnp.int32(CAP))

        n0 = cnt
        n = jnp.minimum(n0 + n1, jnp.int32(CAP))
        m1 = n - n0
        nv0 = lax.shift_right_arithmetic(n0 + 15, 4)
        nv1 = lax.shift_right_arithmetic(m1 + 15, 4)
        nv = lax.shift_right_arithmetic(n + 15, 4)

        def sigkey(v):
            p = 1.0 / (1.0 + jnp.exp(-v))
            return lax.bitcast_convert_type(p, jnp.int32)

        def prob_body(k, _):
            v = cval[pl.ds(k * L, L)]
            kkey = sigkey(v)
            rem = n0 - k * L
            mk = lanes < rem
            pk_a[pl.ds(k * L, L)] = jnp.where(mk, kkey, 0)
            pi_a[pl.ds(k * L, L)] = jnp.where(mk, cidx[pl.ds(k * L, L)], 0)
            return 0
        lax.fori_loop(0, nv0, prob_body, 0)

        pltpu.sync_copy(sh_cv.at[img_local], cval)
        pltpu.sync_copy(sh_ci.at[img_local], cidx)

        def prob2_body(k, _):
            v = cval[pl.ds(k * L, L)]
            kkey = sigkey(v)
            rem = m1 - k * L
            mk = lanes < rem
            cc = n0 + k * L
            plsc.store_compressed(pk_a.at[pl.ds(cc, L)],
                                  jnp.where(mk, kkey, 0), mask=mk)
            plsc.store_compressed(pi_a.at[pl.ds(cc, L)],
                                  jnp.where(mk, cidx[pl.ds(k * L, L)], 0),
                                  mask=mk)
            return 0
        lax.fori_loop(0, nv1, prob2_body, 0)

        pk_a[pl.ds(n, L)] = jnp.zeros((L,), jnp.int32)
        pi_a[pl.ds(n, L)] = jnp.zeros((L,), jnp.int32)

        for t in range(32):
            if t % 2 == 0:
                src_k, src_i, dst_k, dst_i = pk_a, pi_a, pk_b, pi_b
            else:
                src_k, src_i, dst_k, dst_i = pk_b, pi_b, pk_a, pi_a
            bitmask = jnp.int32(np.int32(np.uint32(1 << t)))

            def cnt_body(k, c1):
                kk = src_k[pl.ds(k * L, L)]
                mb = lax.bitwise_and(kk, bitmask) != 0
                return c1 + mb.astype(jnp.int32)
            ones_vec = lax.fori_loop(0, nv, cnt_body,
                                     jnp.zeros((L,), jnp.int32))
            n_ones = jnp.sum(ones_vec)

            def part_body(k, carry):
                oc, zc = carry
                kk = src_k[pl.ds(k * L, L)]
                ii = src_i[pl.ds(k * L, L)]
                mb = lax.bitwise_and(kk, bitmask) != 0
                mz = jnp.logical_not(mb)
                plsc.store_compressed(dst_k.at[pl.ds(oc, L)], kk, mask=mb)
                plsc.store_compressed(dst_i.at[pl.ds(oc, L)], ii, mask=mb)
                plsc.store_compressed(dst_k.at[pl.ds(zc, L)], kk, mask=mz)
                plsc.store_compressed(dst_i.at[pl.ds(zc, L)], ii, mask=mz)
                no = jnp.sum(mb.astype(jnp.int32))
                return oc + no, zc + (L - no)
            lax.fori_loop(0, nv, part_body, (jnp.int32(0), n_ones))

        pltpu.make_async_copy(boxes_hbm.at[img], boxes_vm, sem_box).wait()
        tsv = ts_vm[pl.ds(img * 2, L)]
        fh = tsv[0].astype(jnp.float32)
        fw = tsv[1].astype(jnp.float32)
        inv91 = jnp.float32(1.0 / 91.0)

        for j in range(OUTP // L):
            kk = pk_a[pl.ds(j * L, L)]
            sc_buf[pl.ds(j * L, L)] = lax.bitcast_convert_type(kk,
                                                               jnp.float32)
            iv = pi_a[pl.ds(j * L, L)]
            fidx = iv.astype(jnp.float32)
            bidx = ((fidx + 0.5) * inv91).astype(jnp.int32)
            lbl = iv - bidx * 91
            bidx = jnp.clip(bidx, 0, N_BOX - 1)
            lb_buf[pl.ds(j * L, L)] = jnp.clip(lbl, 0, N_CLS - 1)
            b4 = lax.shift_left(bidx, 2)
            cx = plsc.load_gather(boxes_vm, [b4])
            cy = plsc.load_gather(boxes_vm, [b4 + 1])
            w_ = plsc.load_gather(boxes_vm, [b4 + 2])
            h_ = plsc.load_gather(boxes_vm, [b4 + 3])
            x0 = (cx - 0.5 * w_) * fw
            y0 = (cy - 0.5 * h_) * fh
            x1 = (cx + 0.5 * w_) * fw
            y1 = (cy + 0.5 * h_) * fh
            pos = lax.shift_left(lanes + j * L, 2)
            plsc.store_scatter(bx_buf, [pos], x0)
            plsc.store_scatter(bx_buf, [pos + 1], y0)
            plsc.store_scatter(bx_buf, [pos + 2], x1)
            plsc.store_scatter(bx_buf, [pos + 3], y1)

        pltpu.sync_copy(sc_buf, scores_hbm.at[img])
        pltpu.sync_copy(lb_buf, labels_hbm.at[img])
        pltpu.sync_copy(bx_buf, boxesf_hbm.at[img])


@jax.jit
def _postprocess_sc(x, xtail, boxes, ts):
    mesh = plsc.VectorSubcoreMesh(core_axis_name="c", subcore_axis_name="s")
    f = pl.kernel(
        _sc_body,
        out_type=(
            jax.ShapeDtypeStruct((B, OUTP), jnp.float32),
            jax.ShapeDtypeStruct((B, OUTP), jnp.int32),
            jax.ShapeDtypeStruct((B, OUTP * 4), jnp.float32),
        ),
        mesh=mesh,
        compiler_params=pltpu.CompilerParams(needs_layout_passes=False),
        scratch_types=[
            pltpu.VMEM((2, WIN), jnp.float32),
            pltpu.VMEM((NBINS * L,), jnp.int32),
            pltpu.VMEM((2048,), jnp.int32),
            pltpu.VMEM((CAPB,), jnp.float32),
            pltpu.VMEM((CAPB,), jnp.int32),
            pltpu.VMEM((CAP + L,), jnp.int32),
            pltpu.VMEM((CAP + L,), jnp.int32),
            pltpu.VMEM((CAP + L,), jnp.int32),
            pltpu.VMEM((CAP + L,), jnp.int32),
            pltpu.VMEM((BOXW,), jnp.float32),
            pltpu.VMEM((OUTP,), jnp.float32),
            pltpu.VMEM((OUTP,), jnp.int32),
            pltpu.VMEM((OUTP * 4,), jnp.float32),
            pltpu.VMEM((128 + L,), jnp.int32),
            pltpu.VMEM((128,), jnp.int32),
            pltpu.VMEM_SHARED((IPC, NBINS * L), jnp.int32),
            pltpu.VMEM_SHARED((IPC, CAPB), jnp.float32),
            pltpu.VMEM_SHARED((IPC, CAPB), jnp.int32),
            pltpu.VMEM_SHARED((IPC, 128), jnp.int32),
            pltpu.VMEM_SHARED((IPC, 128), jnp.int32),
            pltpu.SemaphoreType.DMA,
            pltpu.SemaphoreType.DMA((2,)),
        ],
    )
    return f(x, xtail, boxes, ts)


def kernel(pred_logits, pred_boxes, target_sizes):
    x = pred_logits.reshape(B, TOT)
    xtail = jnp.pad(x[:, NFULL * WIN:], ((0, 0), (0, TAIL_P - TAIL)),
                    constant_values=-jnp.inf)
    boxes_p = jnp.pad(pred_boxes.reshape(B, N_BOX * 4),
                      ((0, 0), (0, BOXW - N_BOX * 4)))
    ts = jnp.pad(target_sizes.astype(jnp.int32).reshape(2 * B),
                 (0, 128 - 2 * B))
    scores_p, labels_p, boxes_f = _postprocess_sc(x, xtail, boxes_p, ts)
    return (scores_p[:, :NUM_SEL], labels_p[:, :NUM_SEL],
            boxes_f.reshape(B, OUTP, 4)[:, :NUM_SEL])
